# Initial kernel scaffold; baseline (speedup 1.0000x reference)
import jax
import jax.numpy as jnp
from jax.experimental import pallas as pl


def kernel(points, transform, sphere_centers, cov_inv, norm_factor, opacities):
    raise NotImplementedError("write your pallas kernel here")


# speedup vs baseline: 12.6686x; 12.6686x over previous
import functools

import jax
import jax.numpy as jnp
from jax.experimental import pallas as pl
from jax.experimental.pallas import tpu as pltpu

_TOP_K = 8
_N_POINTS = 20000
_N_SPHERES = 4096
_BLOCK = 256


def _nll_kernel(tref, pts_ref, w_ref, out_ref, *, nblocks, block, n_points,
                n_spheres, top_k):
    i = pl.program_id(0)

    x0 = pts_ref[:, 0:1]
    y0 = pts_ref[:, 1:2]
    z0 = pts_ref[:, 2:3]
    x = x0 * tref[0, 0] + y0 * tref[0, 1] + z0 * tref[0, 2] + tref[0, 3]
    y = x0 * tref[1, 0] + y0 * tref[1, 1] + z0 * tref[1, 2] + tref[1, 3]
    z = x0 * tref[2, 0] + y0 * tref[2, 1] + z0 * tref[2, 2] + tref[2, 3]

    ones = jnp.ones_like(x)
    feats = jnp.concatenate(
        [x * x, y * y, z * z, x * y, x * z, y * z, x, y, z, ones], axis=1)

    both = jnp.dot(feats, w_ref[...], preferred_element_type=jnp.float32)
    d2 = both[:, :n_spheres]
    s = both[:, n_spheres:]

    col = jax.lax.broadcasted_iota(jnp.int32, (block, n_spheres), 1)

    picked = []
    d2w = d2
    for _ in range(top_k):
        dmin = jnp.min(d2w, axis=1, keepdims=True)
        at_min = d2w == dmin
        first = jnp.min(jnp.where(at_min, col, n_spheres), axis=1,
                        keepdims=True)
        sel = col == first
        picked.append(jnp.sum(jnp.where(sel, s, 0.0), axis=1, keepdims=True))
        d2w = jnp.where(sel, jnp.float32(jnp.inf), d2w)

    sel_s = jnp.concatenate(picked, axis=1)
    m = jnp.max(sel_s, axis=1, keepdims=True)
    ll = m + jnp.log(jnp.sum(jnp.exp(sel_s - m), axis=1, keepdims=True))
    nll = -ll

    row = jax.lax.broadcasted_iota(jnp.int32, (block, 1), 0)
    valid = (i * block + row) < n_points
    psum = jnp.sum(jnp.where(valid, nll, 0.0), keepdims=True)

    @pl.when(i == 0)
    def _():
        out_ref[...] = jnp.zeros_like(out_ref)

    out_ref[...] += psum

    @pl.when(i == nblocks - 1)
    def _():
        out_ref[...] = out_ref[...] / n_points


def kernel(points, transform, sphere_centers, cov_inv, norm_factor, opacities):
    n, k, m = _N_POINTS, _TOP_K, _N_SPHERES
    block = _BLOCK
    nblocks = pl.cdiv(n, block)
    n_pad = nblocks * block

    pts = jnp.pad(points, ((0, n_pad - n), (0, 0)))

    c = cov_inv
    mu = sphere_centers
    cmu = jnp.einsum('mij,mj->mi', c, mu)
    mucmu = jnp.einsum('mi,mi->m', cmu, mu)
    log_norm = jnp.log(jnp.clip(norm_factor, 1e-10, None))
    log_op = jnp.log(jnp.clip(opacities, 1e-10, None))

    wd = jnp.stack([
        jnp.ones((m,), jnp.float32),
        jnp.ones((m,), jnp.float32),
        jnp.ones((m,), jnp.float32),
        jnp.zeros((m,), jnp.float32),
        jnp.zeros((m,), jnp.float32),
        jnp.zeros((m,), jnp.float32),
        -2.0 * mu[:, 0],
        -2.0 * mu[:, 1],
        -2.0 * mu[:, 2],
        jnp.sum(mu * mu, axis=1),
    ], axis=0)
    ws = jnp.stack([
        -0.5 * c[:, 0, 0],
        -0.5 * c[:, 1, 1],
        -0.5 * c[:, 2, 2],
        -0.5 * (c[:, 0, 1] + c[:, 1, 0]),
        -0.5 * (c[:, 0, 2] + c[:, 2, 0]),
        -0.5 * (c[:, 1, 2] + c[:, 2, 1]),
        cmu[:, 0],
        cmu[:, 1],
        cmu[:, 2],
        -0.5 * mucmu + log_norm + log_op,
    ], axis=0)
    w = jnp.concatenate([wd, ws], axis=1)

    body = functools.partial(_nll_kernel, nblocks=nblocks, block=block,
                             n_points=n, n_spheres=m, top_k=k)
    out = pl.pallas_call(
        body,
        grid=(nblocks,),
        in_specs=[
            pl.BlockSpec(memory_space=pltpu.SMEM),
            pl.BlockSpec((block, 3), lambda i: (i, 0)),
            pl.BlockSpec((10, 2 * m), lambda i: (0, 0)),
        ],
        out_specs=pl.BlockSpec((1, 1), lambda i: (0, 0)),
        out_shape=jax.ShapeDtypeStruct((1, 1), jnp.float32),
    )(transform, pts, w)
    return out[0, 0]


# speedup vs baseline: 26.2368x; 2.0710x over previous
import functools

import jax
import jax.numpy as jnp
from jax.experimental import pallas as pl
from jax.experimental.pallas import tpu as pltpu

_TOP_K = 8
_N_POINTS = 20000
_N_SPHERES = 4096
_BLOCK = 256


def _nll_kernel(tref, pts_ref, w_ref, out_ref, *, nblocks, block, n_points,
                n_spheres, top_k):
    i = pl.program_id(0)

    x0 = pts_ref[:, 0:1]
    y0 = pts_ref[:, 1:2]
    z0 = pts_ref[:, 2:3]
    x = x0 * tref[0, 0] + y0 * tref[0, 1] + z0 * tref[0, 2] + tref[0, 3]
    y = x0 * tref[1, 0] + y0 * tref[1, 1] + z0 * tref[1, 2] + tref[1, 3]
    z = x0 * tref[2, 0] + y0 * tref[2, 1] + z0 * tref[2, 2] + tref[2, 3]

    ones = jnp.ones_like(x)
    feats = jnp.concatenate(
        [x * x, y * y, z * z, x * y, x * z, y * z, x, y, z, ones], axis=1)

    both = jnp.dot(feats, w_ref[...], preferred_element_type=jnp.float32)
    d2 = both[:, :n_spheres]
    s = both[:, n_spheres:]

    d2w = d2
    for _ in range(top_k):
        dmin = jnp.min(d2w, axis=1, keepdims=True)
        d2w = jnp.where(d2w == dmin, jnp.float32(jnp.inf), d2w)

    sel = d2w != d2
    ms = jnp.max(jnp.where(sel, s, jnp.float32(-1e30)), axis=1,
                 keepdims=True)
    tot = jnp.sum(jnp.where(sel, jnp.exp(jnp.minimum(s - ms, 0.0)), 0.0),
                  axis=1, keepdims=True)
    nll = -(ms + jnp.log(tot))

    row = jax.lax.broadcasted_iota(jnp.int32, (block, 1), 0)
    valid = (i * block + row) < n_points
    psum = jnp.sum(jnp.where(valid, nll, 0.0), keepdims=True)

    @pl.when(i == 0)
    def _():
        out_ref[...] = jnp.zeros_like(out_ref)

    out_ref[...] += psum

    @pl.when(i == nblocks - 1)
    def _():
        out_ref[...] = out_ref[...] / n_points


def kernel(points, transform, sphere_centers, cov_inv, norm_factor, opacities):
    n, k, m = _N_POINTS, _TOP_K, _N_SPHERES
    block = _BLOCK
    nblocks = pl.cdiv(n, block)
    n_pad = nblocks * block

    pts = jnp.pad(points, ((0, n_pad - n), (0, 0)))

    c = cov_inv
    mu = sphere_centers
    cmu = jnp.einsum('mij,mj->mi', c, mu)
    mucmu = jnp.einsum('mi,mi->m', cmu, mu)
    log_norm = jnp.log(jnp.clip(norm_factor, 1e-10, None))
    log_op = jnp.log(jnp.clip(opacities, 1e-10, None))

    wd = jnp.stack([
        jnp.ones((m,), jnp.float32),
        jnp.ones((m,), jnp.float32),
        jnp.ones((m,), jnp.float32),
        jnp.zeros((m,), jnp.float32),
        jnp.zeros((m,), jnp.float32),
        jnp.zeros((m,), jnp.float32),
        -2.0 * mu[:, 0],
        -2.0 * mu[:, 1],
        -2.0 * mu[:, 2],
        jnp.sum(mu * mu, axis=1),
    ], axis=0)
    ws = jnp.stack([
        -0.5 * c[:, 0, 0],
        -0.5 * c[:, 1, 1],
        -0.5 * c[:, 2, 2],
        -0.5 * (c[:, 0, 1] + c[:, 1, 0]),
        -0.5 * (c[:, 0, 2] + c[:, 2, 0]),
        -0.5 * (c[:, 1, 2] + c[:, 2, 1]),
        cmu[:, 0],
        cmu[:, 1],
        cmu[:, 2],
        -0.5 * mucmu + log_norm + log_op,
    ], axis=0)
    w = jnp.concatenate([wd, ws], axis=1)

    body = functools.partial(_nll_kernel, nblocks=nblocks, block=block,
                             n_points=n, n_spheres=m, top_k=k)
    out = pl.pallas_call(
        body,
        grid=(nblocks,),
        in_specs=[
            pl.BlockSpec(memory_space=pltpu.SMEM),
            pl.BlockSpec((block, 3), lambda i: (i, 0)),
            pl.BlockSpec((10, 2 * m), lambda i: (0, 0)),
        ],
        out_specs=pl.BlockSpec((1, 1), lambda i: (0, 0)),
        out_shape=jax.ShapeDtypeStruct((1, 1), jnp.float32),
    )(transform, pts, w)
    return out[0, 0]


# speedup vs baseline: 32.2575x; 1.2295x over previous
import functools

import jax
import jax.numpy as jnp
from jax.experimental import pallas as pl
from jax.experimental.pallas import tpu as pltpu

_TOP_K = 8
_N_POINTS = 20000
_N_SPHERES = 4096
_BLOCK = 512


def _nll_kernel(tref, pts_ref, w_ref, out_ref, *, nblocks, block, n_points,
                n_spheres, top_k):
    i = pl.program_id(0)

    x0 = pts_ref[0:1, :]
    y0 = pts_ref[1:2, :]
    z0 = pts_ref[2:3, :]
    x = x0 * tref[0, 0] + y0 * tref[0, 1] + z0 * tref[0, 2] + tref[0, 3]
    y = x0 * tref[1, 0] + y0 * tref[1, 1] + z0 * tref[1, 2] + tref[1, 3]
    z = x0 * tref[2, 0] + y0 * tref[2, 1] + z0 * tref[2, 2] + tref[2, 3]

    feats = jnp.concatenate(
        [x * x, y * y, z * z, x * y, x * z, y * z, x, y, z,
         jnp.ones_like(x)], axis=0)

    both = jax.lax.dot_general(feats, w_ref[...],
                               (((0,), (0,)), ((), ())),
                               preferred_element_type=jnp.float32)
    d2 = both[:, :n_spheres]
    s = both[:, n_spheres:]

    v = jnp.min(d2, axis=1, keepdims=True)
    for _ in range(top_k - 1):
        v = jnp.min(jnp.where(d2 > v, d2, jnp.float32(jnp.inf)), axis=1,
                    keepdims=True)

    sel = d2 <= v
    ms = jnp.max(jnp.where(sel, s, jnp.float32(-1e30)), axis=1,
                 keepdims=True)
    tot = jnp.sum(jnp.where(sel, jnp.exp(s - ms), 0.0), axis=1,
                  keepdims=True)
    nll = -(ms + jnp.log(tot))

    row = jax.lax.broadcasted_iota(jnp.int32, (block, 1), 0)
    valid = (i * block + row) < n_points
    psum = jnp.sum(jnp.where(valid, nll, 0.0), keepdims=True)

    @pl.when(i == 0)
    def _():
        out_ref[...] = jnp.zeros_like(out_ref)

    out_ref[...] += psum

    @pl.when(i == nblocks - 1)
    def _():
        out_ref[...] = out_ref[...] / n_points


def kernel(points, transform, sphere_centers, cov_inv, norm_factor, opacities):
    n, k, m = _N_POINTS, _TOP_K, _N_SPHERES
    block = _BLOCK
    nblocks = pl.cdiv(n, block)
    n_pad = nblocks * block

    pts_t = jnp.pad(points, ((0, n_pad - n), (0, 0))).T

    c = cov_inv
    mu = sphere_centers
    cmu = jnp.einsum('mij,mj->mi', c, mu)
    mucmu = jnp.einsum('mi,mi->m', cmu, mu)
    log_norm = jnp.log(jnp.clip(norm_factor, 1e-10, None))
    log_op = jnp.log(jnp.clip(opacities, 1e-10, None))

    wd = jnp.stack([
        jnp.ones((m,), jnp.float32),
        jnp.ones((m,), jnp.float32),
        jnp.ones((m,), jnp.float32),
        jnp.zeros((m,), jnp.float32),
        jnp.zeros((m,), jnp.float32),
        jnp.zeros((m,), jnp.float32),
        -2.0 * mu[:, 0],
        -2.0 * mu[:, 1],
        -2.0 * mu[:, 2],
        jnp.sum(mu * mu, axis=1),
    ], axis=0)
    ws = jnp.stack([
        -0.5 * c[:, 0, 0],
        -0.5 * c[:, 1, 1],
        -0.5 * c[:, 2, 2],
        -0.5 * (c[:, 0, 1] + c[:, 1, 0]),
        -0.5 * (c[:, 0, 2] + c[:, 2, 0]),
        -0.5 * (c[:, 1, 2] + c[:, 2, 1]),
        cmu[:, 0],
        cmu[:, 1],
        cmu[:, 2],
        -0.5 * mucmu + log_norm + log_op,
    ], axis=0)
    w = jnp.concatenate([wd, ws], axis=1)

    body = functools.partial(_nll_kernel, nblocks=nblocks, block=block,
                             n_points=n, n_spheres=m, top_k=k)
    out = pl.pallas_call(
        body,
        grid=(nblocks,),
        in_specs=[
            pl.BlockSpec(memory_space=pltpu.SMEM),
            pl.BlockSpec((3, block), lambda i: (0, i)),
            pl.BlockSpec((10, 2 * m), lambda i: (0, 0)),
        ],
        out_specs=pl.BlockSpec((1, 1), lambda i: (0, 0)),
        out_shape=jax.ShapeDtypeStruct((1, 1), jnp.float32),
    )(transform, pts_t, w)
    return out[0, 0]
